# baseline (speedup 1.0000x reference)
import jax
import jax.numpy as jnp
from jax import lax
from jax.experimental import pallas as pl
from jax.experimental.pallas import tpu as pltpu
from jax.experimental.pallas import tpu_sc as plsc

DIM_E = 64
BATCH = 1024
LANES = 128

_SC_INFO = plsc.get_sparse_core_info()
_NC = _SC_INFO.num_cores
_NS = _SC_INFO.num_subcores
_NW = _NC * _NS
_B_PER_W = BATCH // _NW
_WAVE = 4
_DEPTH = 3
_NWAVES = _B_PER_W // _WAVE


def _gather_body(table_hbm, idx_hbm, out_hbm, idx_v, block_v, rows_v,
                 sem0, sem1, sem2):
    wid = lax.axis_index("s") * _NC + lax.axis_index("c")
    base = wid * _B_PER_W
    pltpu.sync_copy(idx_hbm.at[pl.ds(base, _B_PER_W)], idx_v)
    vecs = [idx_v[pl.ds(g * 16, 16)] for g in range(_B_PER_W // 16)]
    sems = [sem0, sem1, sem2]

    def fire(w):
        descs = []
        for i in range(_WAVE):
            k = w * _WAVE + i
            j = vecs[k // 16][k % 16]
            jb = pl.multiple_of((j >> 7) << 7, LANES)
            descs.append(
                pltpu.async_copy(
                    table_hbm.at[:, pl.ds(jb, LANES)],
                    block_v.at[(w % _DEPTH) * _WAVE + i],
                    sems[w % _DEPTH],
                )
            )
        return descs

    def select(w, descs):
        for d in descs:
            d.wait()
        for i in range(_WAVE):
            k = w * _WAVE + i
            j = vecs[k // 16][k % 16]
            cvec = jnp.full((16,), j & (LANES - 1), dtype=jnp.int32)
            kvec = jnp.full((16,), k, dtype=jnp.int32)
            blk = block_v.at[(w % _DEPTH) * _WAVE + i]
            for g in range(DIM_E // 16):
                ridx = lax.iota(jnp.int32, 16) + g * 16
                vals = plsc.load_gather(blk, [ridx, cvec])
                plsc.store_scatter(rows_v, [kvec, ridx], vals)

    pending = {0: fire(0), 1: fire(1)}
    for w in range(_NWAVES):
        if w + 2 < _NWAVES:
            pending[w + 2] = fire(w + 2)
        select(w, pending.pop(w))
    pltpu.sync_copy(rows_v, out_hbm.at[pl.ds(base, _B_PER_W)])


def _sc_gather(table_t, idx):
    mesh = plsc.VectorSubcoreMesh(core_axis_name="c", subcore_axis_name="s")
    return pl.kernel(
        _gather_body,
        mesh=mesh,
        out_type=jax.ShapeDtypeStruct((BATCH, DIM_E), jnp.float32),
        scratch_types=[
            pltpu.VMEM((_B_PER_W,), jnp.int32),
            pltpu.VMEM((_DEPTH * _WAVE, DIM_E, LANES), jnp.float32),
            pltpu.VMEM((_B_PER_W, DIM_E), jnp.float32),
            pltpu.SemaphoreType.DMA,
            pltpu.SemaphoreType.DMA,
            pltpu.SemaphoreType.DMA,
        ],
        compiler_params=pltpu.CompilerParams(needs_layout_passes=False),
    )(table_t, idx)


_RING = 16
_UNROLL = 8


def _tc_gather_body(idx_ref, table_ref, out_ref, ring_ref, sems):
    def start(k):
        j = idx_ref[k]
        jb = pl.multiple_of((j >> 7) << 7, LANES)
        pltpu.make_async_copy(
            table_ref.at[:, pl.ds(jb, LANES)],
            ring_ref.at[k & (_RING - 1)],
            sems.at[k & (_RING - 1)],
        ).start()

    def finish(k):
        slot = k & (_RING - 1)
        pltpu.make_async_copy(
            table_ref.at[:, pl.ds(0, LANES)],
            ring_ref.at[slot],
            sems.at[slot],
        ).wait()
        c = idx_ref[k] % LANES
        onehot = (lax.broadcasted_iota(jnp.int32, (1, LANES), 1) == c)
        row = lax.dot_general(
            onehot.astype(jnp.float32), ring_ref[slot],
            (((1,), (1,)), ((), ())),
            preferred_element_type=jnp.float32,
        )
        out_ref[pl.ds(k, 1)] = row.reshape(1, 1, DIM_E)

    for k in range(_RING):
        start(k)

    def loop_body(o, _):
        for t in range(_UNROLL):
            k = o * _UNROLL + t
            finish(k)

            @pl.when(k + _RING < BATCH)
            def _():
                start(k + _RING)

        return ()

    lax.fori_loop(0, BATCH // _UNROLL, loop_body, ())


def _tc_gather(table_t, idx):
    out = pl.pallas_call(
        _tc_gather_body,
        in_specs=[
            pl.BlockSpec(memory_space=pltpu.SMEM),
            pl.BlockSpec(memory_space=pl.ANY),
        ],
        out_specs=pl.BlockSpec(memory_space=pltpu.VMEM),
        out_shape=jax.ShapeDtypeStruct((BATCH, 1, DIM_E), jnp.float32),
        scratch_shapes=[
            pltpu.VMEM((_RING, DIM_E, LANES), jnp.float32),
            pltpu.SemaphoreType.DMA((_RING,)),
        ],
    )(idx, table_t)
    return out.reshape(BATCH, DIM_E)


def _mm_body(u_ref, v_ref, o_ref):
    o_ref[...] = lax.dot_general(
        u_ref[...], v_ref[...], (((1,), (1,)), ((), ())),
        preferred_element_type=jnp.float32,
    )


def _tc_matmul(u_rows, v_rows):
    return pl.pallas_call(
        _mm_body,
        out_shape=jax.ShapeDtypeStruct((BATCH, BATCH), jnp.float32),
    )(u_rows, v_rows)


def kernel(user_tensor, item_tensor, id_embedding):
    table_t = id_embedding.T
    u_rows = _sc_gather(table_t, user_tensor.astype(jnp.int32))
    v_rows = _tc_gather(table_t, item_tensor.astype(jnp.int32))
    return _tc_matmul(u_rows, v_rows)


# speedup vs baseline: 3.0387x; 3.0387x over previous
import jax
import jax.numpy as jnp
from jax import lax
from jax.experimental import pallas as pl
from jax.experimental.pallas import tpu as pltpu
from jax.experimental.pallas import tpu_sc as plsc

DIM_E = 64
BATCH = 1024
LANES = 128
N_TC = 256
N_SC = 2 * BATCH - N_TC

_SC_INFO = plsc.get_sparse_core_info()
_NC = _SC_INFO.num_cores
_NS = _SC_INFO.num_subcores
_NW = _NC * _NS
_B_PER_W = N_SC // _NW
_U_PER_W = BATCH // _NW
_I_PER_W = _B_PER_W - _U_PER_W
_WAVE = 4
_DEPTH = 3
_NWAVES = _B_PER_W // _WAVE


def _gather_body(table_hbm, user_hbm, item_hbm, out_hbm,
                 idx_v, block_v, rows_v, sem0, sem1, sem2):
    wid = lax.axis_index("s") * _NC + lax.axis_index("c")
    pltpu.sync_copy(user_hbm.at[pl.ds(wid * _U_PER_W, _U_PER_W)],
                    idx_v.at[pl.ds(0, _U_PER_W)])
    pltpu.sync_copy(item_hbm.at[pl.ds(wid * _I_PER_W, _I_PER_W)],
                    idx_v.at[pl.ds(_U_PER_W, _I_PER_W)])
    vecs = [idx_v[pl.ds(g * 16, 16)] for g in range((_B_PER_W + 15) // 16)]
    sems = [sem0, sem1, sem2]

    def fire(w):
        descs = []
        for i in range(_WAVE):
            k = w * _WAVE + i
            j = vecs[k // 16][k % 16]
            jb = pl.multiple_of((j >> 7) << 7, LANES)
            descs.append(
                pltpu.async_copy(
                    table_hbm.at[:, pl.ds(jb, LANES)],
                    block_v.at[(w % _DEPTH) * _WAVE + i],
                    sems[w % _DEPTH],
                )
            )
        return descs

    def select(w, descs):
        for d in descs:
            d.wait()
        for i in range(_WAVE):
            k = w * _WAVE + i
            j = vecs[k // 16][k % 16]
            cvec = jnp.full((16,), j & (LANES - 1), dtype=jnp.int32)
            kvec = jnp.full((16,), k, dtype=jnp.int32)
            blk = block_v.at[(w % _DEPTH) * _WAVE + i]
            for g in range(DIM_E // 16):
                ridx = lax.iota(jnp.int32, 16) + g * 16
                vals = plsc.load_gather(blk, [ridx, cvec])
                plsc.store_scatter(rows_v, [kvec, ridx], vals)

    pending = {0: fire(0), 1: fire(1)}
    for w in range(_NWAVES):
        if w + 2 < _NWAVES:
            pending[w + 2] = fire(w + 2)
        select(w, pending.pop(w))
    pltpu.sync_copy(rows_v.at[pl.ds(0, _U_PER_W)],
                    out_hbm.at[pl.ds(wid * _U_PER_W, _U_PER_W)])
    pltpu.sync_copy(rows_v.at[pl.ds(_U_PER_W, _I_PER_W)],
                    out_hbm.at[pl.ds(BATCH + wid * _I_PER_W, _I_PER_W)])


def _sc_gather(table_t, user_idx, item_idx):
    mesh = plsc.VectorSubcoreMesh(core_axis_name="c", subcore_axis_name="s")
    return pl.kernel(
        _gather_body,
        mesh=mesh,
        out_type=jax.ShapeDtypeStruct((N_SC, DIM_E), jnp.float32),
        scratch_types=[
            pltpu.VMEM((((_B_PER_W + 15) // 16) * 16,), jnp.int32),
            pltpu.VMEM((_DEPTH * _WAVE, DIM_E, LANES), jnp.float32),
            pltpu.VMEM((_B_PER_W, DIM_E), jnp.float32),
            pltpu.SemaphoreType.DMA,
            pltpu.SemaphoreType.DMA,
            pltpu.SemaphoreType.DMA,
        ],
        compiler_params=pltpu.CompilerParams(needs_layout_passes=False),
    )(table_t, user_idx, item_idx)


_TCG = 16
_TC_OFF = BATCH - N_TC


def _tc_gather_body(idx_ref, *refs):
    out_ref = refs[-1]
    blks = refs[:-1]
    i = pl.program_id(0)
    rows = []
    for t in range(_TCG):
        c = idx_ref[_TC_OFF + _TCG * i + t] % LANES
        onehot = (lax.broadcasted_iota(jnp.int32, (1, LANES), 1) == c)
        rows.append(
            lax.dot_general(
                onehot.astype(jnp.float32), blks[t][...],
                (((1,), (1,)), ((), ())),
                preferred_element_type=jnp.float32,
            )
        )
    out_ref[...] = jnp.concatenate(rows, axis=0).reshape(_TCG, 1, DIM_E)


def _tc_gather(table_t, idx):
    grid_spec = pltpu.PrefetchScalarGridSpec(
        num_scalar_prefetch=1,
        grid=(N_TC // _TCG,),
        in_specs=[
            pl.BlockSpec(
                (DIM_E, LANES),
                (lambda t: lambda i, idx_ref:
                 (0, idx_ref[_TC_OFF + _TCG * i + t] // LANES))(t),
            )
            for t in range(_TCG)
        ],
        out_specs=pl.BlockSpec((_TCG, 1, DIM_E), lambda i, idx_ref: (i, 0, 0)),
    )
    out = pl.pallas_call(
        _tc_gather_body,
        grid_spec=grid_spec,
        out_shape=jax.ShapeDtypeStruct((N_TC, 1, DIM_E), jnp.float32),
    )(idx, *([table_t] * _TCG))
    return out.reshape(N_TC, DIM_E)


def _mm_body(sc_ref, tc_ref, o_ref):
    u = sc_ref[0:BATCH, :]
    v1 = sc_ref[BATCH:N_SC, :]
    v2 = tc_ref[...]
    o_ref[:, 0:(N_SC - BATCH)] = lax.dot_general(
        u, v1, (((1,), (1,)), ((), ())), preferred_element_type=jnp.float32
    )
    o_ref[:, (N_SC - BATCH):BATCH] = lax.dot_general(
        u, v2, (((1,), (1,)), ((), ())), preferred_element_type=jnp.float32
    )


def _tc_matmul(sc_rows, tc_rows):
    return pl.pallas_call(
        _mm_body,
        out_shape=jax.ShapeDtypeStruct((BATCH, BATCH), jnp.float32),
    )(sc_rows, tc_rows)


def kernel(user_tensor, item_tensor, id_embedding):
    table_t = id_embedding.T
    user_idx = user_tensor.astype(jnp.int32)
    item_idx = item_tensor.astype(jnp.int32)
    sc_rows = _sc_gather(table_t, user_idx, item_idx)
    tc_rows = _tc_gather(table_t, item_idx)
    return _tc_matmul(sc_rows, tc_rows)


# speedup vs baseline: 3.0538x; 1.0050x over previous
import jax
import jax.numpy as jnp
from jax import lax
from jax.experimental import pallas as pl
from jax.experimental.pallas import tpu as pltpu
from jax.experimental.pallas import tpu_sc as plsc

DIM_E = 64
BATCH = 1024
LANES = 128
N_TC = 256
N_SC = 2 * BATCH - N_TC

_NC = 2
_NS = 16
_NW = _NC * _NS
_B_PER_W = N_SC // _NW
_U_PER_W = BATCH // _NW
_I_PER_W = _B_PER_W - _U_PER_W
_WAVE = 4
_DEPTH = 3
_NWAVES = _B_PER_W // _WAVE


def _gather_body(table_hbm, user_hbm, item_hbm, out_hbm,
                 idx_v, block_v, rows_v, sem0, sem1, sem2):
    wid = lax.axis_index("s") * _NC + lax.axis_index("c")
    pltpu.sync_copy(user_hbm.at[pl.ds(wid * _U_PER_W, _U_PER_W)],
                    idx_v.at[pl.ds(0, _U_PER_W)])
    pltpu.sync_copy(item_hbm.at[pl.ds(wid * _I_PER_W, _I_PER_W)],
                    idx_v.at[pl.ds(_U_PER_W, _I_PER_W)])
    vecs = [idx_v[pl.ds(g * 16, 16)] for g in range((_B_PER_W + 15) // 16)]
    sems = [sem0, sem1, sem2]

    def fire(w):
        descs = []
        for i in range(_WAVE):
            k = w * _WAVE + i
            j = vecs[k // 16][k % 16]
            jb = pl.multiple_of((j >> 7) << 7, LANES)
            descs.append(
                pltpu.async_copy(
                    table_hbm.at[:, pl.ds(jb, LANES)],
                    block_v.at[(w % _DEPTH) * _WAVE + i],
                    sems[w % _DEPTH],
                )
            )
        return descs

    def select(w, descs):
        for d in descs:
            d.wait()
        for i in range(_WAVE):
            k = w * _WAVE + i
            j = vecs[k // 16][k % 16]
            cvec = jnp.full((16,), j & (LANES - 1), dtype=jnp.int32)
            kvec = jnp.full((16,), k, dtype=jnp.int32)
            blk = block_v.at[(w % _DEPTH) * _WAVE + i]
            for g in range(DIM_E // 16):
                ridx = lax.iota(jnp.int32, 16) + g * 16
                vals = plsc.load_gather(blk, [ridx, cvec])
                plsc.store_scatter(rows_v, [kvec, ridx], vals)

    pending = {0: fire(0), 1: fire(1)}
    for w in range(_NWAVES):
        if w + 2 < _NWAVES:
            pending[w + 2] = fire(w + 2)
        select(w, pending.pop(w))
    pltpu.sync_copy(rows_v.at[pl.ds(0, _U_PER_W)],
                    out_hbm.at[pl.ds(wid * _U_PER_W, _U_PER_W)])
    pltpu.sync_copy(rows_v.at[pl.ds(_U_PER_W, _I_PER_W)],
                    out_hbm.at[pl.ds(BATCH + wid * _I_PER_W, _I_PER_W)])


def _sc_gather(table_t, user_idx, item_idx):
    mesh = plsc.VectorSubcoreMesh(core_axis_name="c", subcore_axis_name="s")
    return pl.kernel(
        _gather_body,
        mesh=mesh,
        out_type=jax.ShapeDtypeStruct((N_SC, DIM_E), jnp.float32),
        scratch_types=[
            pltpu.VMEM((((_B_PER_W + 15) // 16) * 16,), jnp.int32),
            pltpu.VMEM((_DEPTH * _WAVE, DIM_E, LANES), jnp.float32),
            pltpu.VMEM((_B_PER_W, DIM_E), jnp.float32),
            pltpu.SemaphoreType.DMA,
            pltpu.SemaphoreType.DMA,
            pltpu.SemaphoreType.DMA,
        ],
        compiler_params=pltpu.CompilerParams(needs_layout_passes=False),
    )(table_t, user_idx, item_idx)


_TCG = 16
_TC_OFF = BATCH - N_TC


def _tc_gather_body(idx_ref, *refs):
    out_ref = refs[-1]
    blks = refs[:-1]
    i = pl.program_id(0)
    rows = []
    for t in range(_TCG):
        c = idx_ref[_TC_OFF + _TCG * i + t] % LANES
        onehot = (lax.broadcasted_iota(jnp.int32, (1, LANES), 1) == c)
        rows.append(
            lax.dot_general(
                onehot.astype(jnp.float32), blks[t][...],
                (((1,), (1,)), ((), ())),
                preferred_element_type=jnp.float32,
            )
        )
    out_ref[...] = jnp.concatenate(rows, axis=0).reshape(_TCG, 1, DIM_E)


def _tc_gather(table_t, idx):
    grid_spec = pltpu.PrefetchScalarGridSpec(
        num_scalar_prefetch=1,
        grid=(N_TC // _TCG,),
        in_specs=[
            pl.BlockSpec(
                (DIM_E, LANES),
                (lambda t: lambda i, idx_ref:
                 (0, idx_ref[_TC_OFF + _TCG * i + t] // LANES))(t),
            )
            for t in range(_TCG)
        ],
        out_specs=pl.BlockSpec((_TCG, 1, DIM_E), lambda i, idx_ref: (i, 0, 0)),
    )
    out = pl.pallas_call(
        _tc_gather_body,
        grid_spec=grid_spec,
        out_shape=jax.ShapeDtypeStruct((N_TC, 1, DIM_E), jnp.float32),
    )(idx, *([table_t] * _TCG))
    return out.reshape(N_TC, DIM_E)


def _mm_body(sc_ref, tc_ref, o_ref):
    u = sc_ref[0:BATCH, :]
    v1 = sc_ref[BATCH:N_SC, :]
    v2 = tc_ref[...]
    o_ref[:, 0:(N_SC - BATCH)] = lax.dot_general(
        u, v1, (((1,), (1,)), ((), ())), preferred_element_type=jnp.float32
    )
    o_ref[:, (N_SC - BATCH):BATCH] = lax.dot_general(
        u, v2, (((1,), (1,)), ((), ())), preferred_element_type=jnp.float32
    )


def _tc_matmul(sc_rows, tc_rows):
    return pl.pallas_call(
        _mm_body,
        out_shape=jax.ShapeDtypeStruct((BATCH, BATCH), jnp.float32),
    )(sc_rows, tc_rows)


def kernel(user_tensor, item_tensor, id_embedding):
    table_t = id_embedding.T
    user_idx = user_tensor.astype(jnp.int32)
    item_idx = item_tensor.astype(jnp.int32)
    sc_rows = _sc_gather(table_t, user_idx, item_idx)
    tc_rows = _tc_gather(table_t, item_idx)
    return _tc_matmul(sc_rows, tc_rows)


# speedup vs baseline: 3.1482x; 1.0309x over previous
import jax
import jax.numpy as jnp
from jax import lax
from jax.experimental import pallas as pl
from jax.experimental.pallas import tpu as pltpu
from jax.experimental.pallas import tpu_sc as plsc

DIM_E = 64
BATCH = 1024
LANES = 128
N_TC = 256
N_SC = 2 * BATCH - N_TC

_NC = 2
_NS = 16
_NW = _NC * _NS
_B_PER_W = N_SC // _NW
_U_PER_W = BATCH // _NW
_I_PER_W = _B_PER_W - _U_PER_W
_WAVE = 7
_DEPTH = 2
_NWAVES = _B_PER_W // _WAVE


def _gather_body(table_hbm, user_hbm, item_hbm, out_hbm,
                 idx_v, block_v, rows_v, sem0, sem1, sem2):
    wid = lax.axis_index("s") * _NC + lax.axis_index("c")
    pltpu.sync_copy(user_hbm.at[pl.ds(wid * _U_PER_W, _U_PER_W)],
                    idx_v.at[pl.ds(0, _U_PER_W)])
    pltpu.sync_copy(item_hbm.at[pl.ds(wid * _I_PER_W, _I_PER_W)],
                    idx_v.at[pl.ds(_U_PER_W, _I_PER_W)])
    vecs = [idx_v[pl.ds(g * 16, 16)] for g in range((_B_PER_W + 15) // 16)]
    sems = [sem0, sem1, sem2]

    def fire(w):
        descs = []
        for i in range(_WAVE):
            k = w * _WAVE + i
            j = vecs[k // 16][k % 16]
            jb = pl.multiple_of((j >> 7) << 7, LANES)
            descs.append(
                pltpu.async_copy(
                    table_hbm.at[:, pl.ds(jb, LANES)],
                    block_v.at[(w % _DEPTH) * _WAVE + i],
                    sems[w % _DEPTH],
                )
            )
        return descs

    def select(w, descs):
        for d in descs:
            d.wait()
        for i in range(_WAVE):
            k = w * _WAVE + i
            j = vecs[k // 16][k % 16]
            cvec = jnp.full((16,), j & (LANES - 1), dtype=jnp.int32)
            kvec = jnp.full((16,), k, dtype=jnp.int32)
            blk = block_v.at[(w % _DEPTH) * _WAVE + i]
            for g in range(DIM_E // 16):
                ridx = lax.iota(jnp.int32, 16) + g * 16
                vals = plsc.load_gather(blk, [ridx, cvec])
                plsc.store_scatter(rows_v, [kvec, ridx], vals)

    pending = {0: fire(0), 1: fire(1)}
    for w in range(_NWAVES):
        if w + 2 < _NWAVES:
            pending[w + 2] = fire(w + 2)
        select(w, pending.pop(w))
    pltpu.sync_copy(rows_v.at[pl.ds(0, _U_PER_W)],
                    out_hbm.at[pl.ds(wid * _U_PER_W, _U_PER_W)])
    pltpu.sync_copy(rows_v.at[pl.ds(_U_PER_W, _I_PER_W)],
                    out_hbm.at[pl.ds(BATCH + wid * _I_PER_W, _I_PER_W)])


def _sc_gather(table_t, user_idx, item_idx):
    mesh = plsc.VectorSubcoreMesh(core_axis_name="c", subcore_axis_name="s")
    return pl.kernel(
        _gather_body,
        mesh=mesh,
        out_type=jax.ShapeDtypeStruct((N_SC, DIM_E), jnp.float32),
        scratch_types=[
            pltpu.VMEM((((_B_PER_W + 15) // 16) * 16,), jnp.int32),
            pltpu.VMEM((_DEPTH * _WAVE, DIM_E, LANES), jnp.float32),
            pltpu.VMEM((_B_PER_W, DIM_E), jnp.float32),
            pltpu.SemaphoreType.DMA,
            pltpu.SemaphoreType.DMA,
            pltpu.SemaphoreType.DMA,
        ],
        compiler_params=pltpu.CompilerParams(needs_layout_passes=False),
    )(table_t, user_idx, item_idx)


_TCG = 16
_TC_OFF = BATCH - N_TC


def _tc_gather_body(idx_ref, *refs):
    out_ref = refs[-1]
    blks = refs[:-1]
    i = pl.program_id(0)
    rows = []
    for t in range(_TCG):
        c = idx_ref[_TC_OFF + _TCG * i + t] % LANES
        onehot = (lax.broadcasted_iota(jnp.int32, (1, LANES), 1) == c)
        rows.append(
            lax.dot_general(
                onehot.astype(jnp.float32), blks[t][...],
                (((1,), (1,)), ((), ())),
                preferred_element_type=jnp.float32,
            )
        )
    out_ref[...] = jnp.concatenate(rows, axis=0).reshape(_TCG, 1, DIM_E)


def _tc_gather(table_t, idx):
    grid_spec = pltpu.PrefetchScalarGridSpec(
        num_scalar_prefetch=1,
        grid=(N_TC // _TCG,),
        in_specs=[
            pl.BlockSpec(
                (DIM_E, LANES),
                (lambda t: lambda i, idx_ref:
                 (0, idx_ref[_TC_OFF + _TCG * i + t] // LANES))(t),
            )
            for t in range(_TCG)
        ],
        out_specs=pl.BlockSpec((_TCG, 1, DIM_E), lambda i, idx_ref: (i, 0, 0)),
    )
    out = pl.pallas_call(
        _tc_gather_body,
        grid_spec=grid_spec,
        out_shape=jax.ShapeDtypeStruct((N_TC, 1, DIM_E), jnp.float32),
    )(idx, *([table_t] * _TCG))
    return out.reshape(N_TC, DIM_E)


def _mm_body(sc_ref, tc_ref, o_ref):
    u = sc_ref[0:BATCH, :]
    v1 = sc_ref[BATCH:N_SC, :]
    v2 = tc_ref[...]
    o_ref[:, 0:(N_SC - BATCH)] = lax.dot_general(
        u, v1, (((1,), (1,)), ((), ())), preferred_element_type=jnp.float32
    )
    o_ref[:, (N_SC - BATCH):BATCH] = lax.dot_general(
        u, v2, (((1,), (1,)), ((), ())), preferred_element_type=jnp.float32
    )


def _tc_matmul(sc_rows, tc_rows):
    return pl.pallas_call(
        _mm_body,
        out_shape=jax.ShapeDtypeStruct((BATCH, BATCH), jnp.float32),
    )(sc_rows, tc_rows)


def kernel(user_tensor, item_tensor, id_embedding):
    table_t = id_embedding.T
    user_idx = user_tensor.astype(jnp.int32)
    item_idx = item_tensor.astype(jnp.int32)
    sc_rows = _sc_gather(table_t, user_idx, item_idx)
    tc_rows = _tc_gather(table_t, item_idx)
    return _tc_matmul(sc_rows, tc_rows)
